# baseline (speedup 1.0000x reference)
import functools

import jax
import jax.numpy as jnp
from jax import lax
from jax.experimental import pallas as pl
from jax.experimental.pallas import tpu as pltpu
from jax.experimental.pallas import tpu_sc as plsc

B, C, H, W = 16, 384, 64, 64
ROW = H * W
NROWS = B * C
NW = 32
ROWS_PER_W = NROWS // NW
CK = 8
NCHUNK = ROWS_PER_W // CK
NBUF = 3
L = 16
VSTEP = ROW // L


def _body(in_hbm, out_hbm, buf, sem_ld, sem_st):
    wid = lax.axis_index("s") * 2 + lax.axis_index("c")
    b = wid // 2
    c0 = (wid % 2) * ROWS_PER_W
    base = b * C

    def fire_load(j):
        src0 = base + C - c0 - (j + 1) * CK
        pltpu.make_async_copy(
            in_hbm.at[pl.ds(src0, CK)], buf.at[j % NBUF], sem_ld
        ).start()

    def wait_load(j):
        pltpu.make_async_copy(
            in_hbm.at[pl.ds(base, CK)], buf.at[j % NBUF], sem_ld
        ).wait()

    def fire_store(j):
        pltpu.make_async_copy(
            buf.at[j % NBUF], out_hbm.at[pl.ds(base + c0 + j * CK, CK)], sem_st
        ).start()

    def wait_store(j):
        pltpu.make_async_copy(
            buf.at[j % NBUF], out_hbm.at[pl.ds(base + c0 + j * CK, CK)], sem_st
        ).wait()

    def reverse(j):
        jb = j % NBUF

        def step(k, carry):
            off = k * L
            for r in range(CK // 2):
                a = buf[jb, r, pl.ds(off, L)]
                z = buf[jb, CK - 1 - r, pl.ds(off, L)]
                buf[jb, r, pl.ds(off, L)] = z
                buf[jb, CK - 1 - r, pl.ds(off, L)] = a
            return carry

        lax.fori_loop(0, VSTEP, step, 0)

    fire_load(0)
    fire_load(1)
    for j in range(NCHUNK):
        wait_load(j)
        reverse(j)
        fire_store(j)
        nxt = j + NBUF - 1
        if nxt < NCHUNK:
            if j >= 1:
                wait_store(j - 1)
            fire_load(nxt)
    for j in range(NCHUNK - NBUF, NCHUNK):
        if j >= 0:
            wait_store(j)


@jax.jit
def kernel(input):
    flat = input.reshape(NROWS, ROW)
    mesh = plsc.VectorSubcoreMesh(core_axis_name="c", subcore_axis_name="s")
    out = pl.kernel(
        _body,
        out_type=jax.ShapeDtypeStruct((NROWS, ROW), jnp.float32),
        mesh=mesh,
        scratch_types=[
            pltpu.VMEM((NBUF, CK, ROW), jnp.float32),
            pltpu.SemaphoreType.DMA,
            pltpu.SemaphoreType.DMA,
        ],
    )(flat)
    return out.reshape(B, C, H, W)


# speedup vs baseline: 5.1169x; 5.1169x over previous
import functools

import jax
import jax.numpy as jnp
from jax import lax
from jax.experimental import pallas as pl
from jax.experimental.pallas import tpu as pltpu
from jax.experimental.pallas import tpu_sc as plsc

B, C, H, W = 16, 384, 64, 64
NR = B * H * W
NW = 32
RPW = NR // NW
CKR = 128
NCHUNK = RPW // CKR
NBUF = 2
L = 16
KSTEP = C // (2 * L)


def _body(in_hbm, out_hbm, buf, sem_ld, sem_st):
    wid = lax.axis_index("s") * 2 + lax.axis_index("c")
    r0 = wid * RPW

    def fire_load(j):
        pltpu.make_async_copy(
            in_hbm.at[pl.ds(r0 + j * CKR, CKR)], buf.at[j % NBUF], sem_ld
        ).start()

    def wait_load(j):
        pltpu.make_async_copy(
            in_hbm.at[pl.ds(r0, CKR)], buf.at[j % NBUF], sem_ld
        ).wait()

    def fire_store(j):
        pltpu.make_async_copy(
            buf.at[j % NBUF], out_hbm.at[pl.ds(r0 + j * CKR, CKR)], sem_st
        ).start()

    def wait_store(j):
        pltpu.make_async_copy(
            buf.at[j % NBUF], out_hbm.at[pl.ds(r0 + j * CKR, CKR)], sem_st
        ).wait()

    def reverse(j):
        jb = j % NBUF

        def row(r, carry):
            for k in range(KSTEP):
                lo = k * L
                hi = C - (k + 1) * L
                a = buf[jb, r, pl.ds(lo, L)]
                z = buf[jb, r, pl.ds(hi, L)]
                buf[jb, r, pl.ds(lo, L)] = lax.rev(z, (0,))
                buf[jb, r, pl.ds(hi, L)] = lax.rev(a, (0,))
            return carry

        lax.fori_loop(0, CKR, row, 0)

    fire_load(0)
    for j in range(NCHUNK):
        wait_load(j)
        reverse(j)
        fire_store(j)
        if j + 1 < NCHUNK:
            if j >= 1:
                wait_store(j - 1)
            fire_load(j + 1)
    wait_store(NCHUNK - 1)


@jax.jit
def kernel(input):
    flat = jnp.transpose(input, (0, 2, 3, 1)).reshape(NR, C)
    mesh = plsc.VectorSubcoreMesh(core_axis_name="c", subcore_axis_name="s")
    out = pl.kernel(
        _body,
        out_type=jax.ShapeDtypeStruct((NR, C), jnp.float32),
        mesh=mesh,
        scratch_types=[
            pltpu.VMEM((NBUF, CKR, C), jnp.float32),
            pltpu.SemaphoreType.DMA,
            pltpu.SemaphoreType.DMA,
        ],
        compiler_params=pltpu.CompilerParams(use_tc_tiling_on_sc=True),
    )(flat)
    return jnp.transpose(out.reshape(B, H, W, C), (0, 3, 1, 2))


# speedup vs baseline: 5.9857x; 1.1698x over previous
import functools

import jax
import jax.numpy as jnp
from jax import lax
from jax.experimental import pallas as pl
from jax.experimental.pallas import tpu as pltpu
from jax.experimental.pallas import tpu_sc as plsc

B, C, H, W = 16, 384, 64, 64
NR = B * H * W
NW = 32
RPW = NR // NW
CKR = 64
NCHUNK = RPW // CKR
NBUF = 4
L = 16
KSTEP = C // (2 * L)


def _body(in_hbm, out_hbm, buf, sem_ld, sem_st):
    wid = lax.axis_index("s") * 2 + lax.axis_index("c")
    r0 = wid * RPW

    def fire_load(j):
        pltpu.make_async_copy(
            in_hbm.at[pl.ds(r0 + j * CKR, CKR)], buf.at[j % NBUF], sem_ld
        ).start()

    def wait_load(j):
        pltpu.make_async_copy(
            in_hbm.at[pl.ds(r0, CKR)], buf.at[j % NBUF], sem_ld
        ).wait()

    def fire_store(j):
        pltpu.make_async_copy(
            buf.at[j % NBUF], out_hbm.at[pl.ds(r0 + j * CKR, CKR)], sem_st
        ).start()

    def wait_store(j):
        pltpu.make_async_copy(
            buf.at[j % NBUF], out_hbm.at[pl.ds(r0 + j * CKR, CKR)], sem_st
        ).wait()

    def reverse(j):
        jb = j % NBUF

        def row(r, carry):
            for k in range(KSTEP):
                lo = k * L
                hi = C - (k + 1) * L
                a = buf[jb, r, pl.ds(lo, L)]
                z = buf[jb, r, pl.ds(hi, L)]
                buf[jb, r, pl.ds(lo, L)] = lax.rev(z, (0,))
                buf[jb, r, pl.ds(hi, L)] = lax.rev(a, (0,))
            return carry

        lax.fori_loop(0, CKR, row, 0)

    for j in range(NBUF - 1):
        fire_load(j)
    for j in range(NCHUNK):
        wait_load(j)
        reverse(j)
        fire_store(j)
        nxt = j + NBUF - 1
        if nxt < NCHUNK:
            if j >= 1:
                wait_store(j - 1)
            fire_load(nxt)
    for j in range(max(0, NCHUNK - NBUF), NCHUNK):
        wait_store(j)


@jax.jit
def kernel(input):
    flat = jnp.transpose(input, (0, 2, 3, 1)).reshape(NR, C)
    mesh = plsc.VectorSubcoreMesh(core_axis_name="c", subcore_axis_name="s")
    out = pl.kernel(
        _body,
        out_type=jax.ShapeDtypeStruct((NR, C), jnp.float32),
        mesh=mesh,
        scratch_types=[
            pltpu.VMEM((NBUF, CKR, C), jnp.float32),
            pltpu.SemaphoreType.DMA,
            pltpu.SemaphoreType.DMA,
        ],
        compiler_params=pltpu.CompilerParams(use_tc_tiling_on_sc=True),
    )(flat)
    return jnp.transpose(out.reshape(B, H, W, C), (0, 3, 1, 2))
